# baseline (speedup 1.0000x reference)
import functools

import jax
import jax.numpy as jnp
from jax import lax
from jax.experimental import pallas as pl
from jax.experimental.pallas import tpu as pltpu
from jax.experimental.pallas import tpu_sc as plsc

_E = 64
_TILE = 32
_NT = 96
_NW = 32
_LANES = 16


def _route_body(x_ref, wr_ref, pos1_ref, pos2_ref, w1_ref, w2_ref, te_ref):
    T = x_ref.shape[0]
    x = x_ref[...]
    logits = lax.dot_general(x, wr_ref[...], (((1,), (1,)), ((), ())),
                             preferred_element_type=jnp.float32)
    m = jnp.max(logits, axis=1, keepdims=True)
    p = jnp.exp(logits - m)
    probs = p / jnp.sum(p, axis=1, keepdims=True)
    lane = lax.broadcasted_iota(jnp.int32, (T, _E), 1)
    m1 = jnp.max(probs, axis=1, keepdims=True)
    i1 = jnp.min(jnp.where(probs == m1, lane, _E), axis=1, keepdims=True)
    probs2 = jnp.where(lane == i1, -jnp.inf, probs)
    m2 = jnp.max(probs2, axis=1, keepdims=True)
    i2 = jnp.min(jnp.where(probs2 == m2, lane, _E), axis=1, keepdims=True)
    s = m1 + m2
    w1_ref[...] = m1 / s
    w2_ref[...] = m2 / s

    M1 = (lane == i1).astype(jnp.float32)
    M2 = (lane == i2).astype(jnp.float32)
    cnt1 = jnp.sum(M1, axis=0, keepdims=True)
    cnt2 = jnp.sum(M2, axis=0, keepdims=True)
    cnt = cnt1 + cnt2
    pcnt = jnp.ceil(cnt / _TILE) * _TILE
    ea = lax.broadcasted_iota(jnp.int32, (_E, _E), 0)
    eb = lax.broadcasted_iota(jnp.int32, (_E, _E), 1)
    U = (ea <= eb).astype(jnp.float32)
    pcum = lax.dot_general(pcnt, U, (((1,), (0,)), ((), ())),
                           preferred_element_type=jnp.float32)
    pex = pcum - pcnt
    ta = lax.broadcasted_iota(jnp.int32, (T, T), 0)
    tb = lax.broadcasted_iota(jnp.int32, (T, T), 1)
    S = (tb < ta).astype(jnp.float32)
    C1 = lax.dot_general(S, M1, (((1,), (0,)), ((), ())),
                         preferred_element_type=jnp.float32)
    C2 = lax.dot_general(S, M2, (((1,), (0,)), ((), ())),
                         preferred_element_type=jnp.float32)
    r1 = jnp.sum(C1 * M1, axis=1, keepdims=True)
    r2 = jnp.sum(C2 * M2, axis=1, keepdims=True)
    pos1 = jnp.sum(pex * M1, axis=1, keepdims=True) + r1
    pos2 = jnp.sum((pex + cnt1) * M2, axis=1, keepdims=True) + r2
    pos1_ref[...] = jnp.round(pos1).astype(jnp.int32)
    pos2_ref[...] = jnp.round(pos2).astype(jnp.int32)

    total = pcum[0:1, _E - 1:_E]
    ti = lax.broadcasted_iota(jnp.int32, (_NT, 1), 0).astype(jnp.float32) * _TILE
    r = jnp.minimum(ti, total - _TILE)
    owner = jnp.sum((pcum <= r).astype(jnp.float32), axis=1, keepdims=True)
    te_ref[...] = jnp.round(owner).astype(jnp.int32)


def _expert_body(te_ref, gx_ref, wg_ref, wu_ref, wd_ref, y_ref):
    del te_ref
    gx = gx_ref[...]
    g = lax.dot_general(gx, wg_ref[0], (((1,), (1,)), ((), ())),
                        preferred_element_type=jnp.float32)
    u = lax.dot_general(gx, wu_ref[0], (((1,), (1,)), ((), ())),
                        preferred_element_type=jnp.float32)
    h = (g * jax.nn.sigmoid(g)) * u
    y_ref[...] = lax.dot_general(h, wd_ref[0], (((1,), (1,)), ((), ())),
                                 preferred_element_type=jnp.float32)


def _make_scatter(T, D, PAD):
    ntok = T // _NW
    mesh = plsc.VectorSubcoreMesh(core_axis_name="c", subcore_axis_name="s")

    @functools.partial(
        pl.kernel, mesh=mesh,
        out_type=jax.ShapeDtypeStruct((PAD, D), jnp.float32),
        scratch_types=[
            pltpu.VMEM((ntok,), jnp.int32),
            pltpu.VMEM((ntok,), jnp.int32),
            pltpu.VMEM((ntok, D), jnp.float32),
            pltpu.SemaphoreType.DMA,
            pltpu.SemaphoreType.DMA,
        ],
    )
    def scatter_k(x_hbm, p1_hbm, p2_hbm, gx_hbm, i1v, i2v, rows, sem1, sem2):
        wid = lax.axis_index("s") * 2 + lax.axis_index("c")
        base = wid * ntok
        pltpu.sync_copy(p1_hbm.at[pl.ds(base, ntok)], i1v)
        pltpu.sync_copy(p2_hbm.at[pl.ds(base, ntok)], i2v)
        pltpu.sync_copy(x_hbm.at[pl.ds(base, ntok)], rows)
        c1 = pltpu.async_copy(rows, gx_hbm.at[i1v], sem1)
        c2 = pltpu.async_copy(rows, gx_hbm.at[i2v], sem2)
        c1.wait()
        c2.wait()

    return scatter_k


def _make_combine(T, D):
    ntok = T // _NW
    nchunk = D // _LANES
    mesh = plsc.VectorSubcoreMesh(core_axis_name="c", subcore_axis_name="s")

    @functools.partial(
        pl.kernel, mesh=mesh,
        out_type=jax.ShapeDtypeStruct((T, D), jnp.float32),
        scratch_types=[
            pltpu.VMEM((ntok,), jnp.int32),
            pltpu.VMEM((ntok,), jnp.int32),
            pltpu.VMEM((ntok,), jnp.float32),
            pltpu.VMEM((ntok,), jnp.float32),
            pltpu.VMEM((ntok, D), jnp.float32),
            pltpu.VMEM((ntok, D), jnp.float32),
            pltpu.SemaphoreType.DMA,
            pltpu.SemaphoreType.DMA,
        ],
    )
    def combine_k(y_hbm, p1_hbm, p2_hbm, wa_hbm, wb_hbm, out_hbm,
                  i1v, i2v, w1v, w2v, buf1, buf2, sem1, sem2):
        wid = lax.axis_index("s") * 2 + lax.axis_index("c")
        base = wid * ntok
        pltpu.sync_copy(p1_hbm.at[pl.ds(base, ntok)], i1v)
        pltpu.sync_copy(p2_hbm.at[pl.ds(base, ntok)], i2v)
        pltpu.sync_copy(wa_hbm.at[pl.ds(base, ntok)], w1v)
        pltpu.sync_copy(wb_hbm.at[pl.ds(base, ntok)], w2v)
        c1 = pltpu.async_copy(y_hbm.at[i1v], buf1, sem1)
        c2 = pltpu.async_copy(y_hbm.at[i2v], buf2, sem2)
        c1.wait()
        c2.wait()
        wvec1 = w1v[...]
        wvec2 = w2v[...]
        for row in range(ntok):
            ws1 = jnp.full((_LANES,), wvec1[row], jnp.float32)
            ws2 = jnp.full((_LANES,), wvec2[row], jnp.float32)

            def body(c, _, row=row, ws1=ws1, ws2=ws2):
                a = buf1[row, pl.ds(c * _LANES, _LANES)]
                bb = buf2[row, pl.ds(c * _LANES, _LANES)]
                buf1[row, pl.ds(c * _LANES, _LANES)] = ws1 * a + ws2 * bb
                return 0

            lax.fori_loop(0, nchunk, body, 0)
        pltpu.sync_copy(buf1, out_hbm.at[pl.ds(base, ntok)])

    return combine_k


def kernel(hidden_states, W_router, W_gate, W_up, W_down):
    b, s, d = hidden_states.shape
    x = hidden_states.reshape(-1, d)
    T = x.shape[0]
    E, FF = W_gate.shape[0], W_gate.shape[1]
    PAD = _NT * _TILE

    pos1, pos2, w1, w2, te = pl.pallas_call(
        _route_body,
        grid=(1,),
        in_specs=[
            pl.BlockSpec((T, d), lambda i: (0, 0)),
            pl.BlockSpec((E, d), lambda i: (0, 0)),
        ],
        out_specs=[
            pl.BlockSpec((T, 1), lambda i: (0, 0)),
            pl.BlockSpec((T, 1), lambda i: (0, 0)),
            pl.BlockSpec((T, 1), lambda i: (0, 0)),
            pl.BlockSpec((T, 1), lambda i: (0, 0)),
            pl.BlockSpec((_NT, 1), lambda i: (0, 0)),
        ],
        out_shape=[
            jax.ShapeDtypeStruct((T, 1), jnp.int32),
            jax.ShapeDtypeStruct((T, 1), jnp.int32),
            jax.ShapeDtypeStruct((T, 1), jnp.float32),
            jax.ShapeDtypeStruct((T, 1), jnp.float32),
            jax.ShapeDtypeStruct((_NT, 1), jnp.int32),
        ],
    )(x, W_router)
    p1 = pos1.reshape(T)
    p2 = pos2.reshape(T)
    wa = w1.reshape(T)
    wb = w2.reshape(T)
    te_arr = te.reshape(_NT)

    gx = _make_scatter(T, d, PAD)(x, p1, p2)

    grid_spec = pltpu.PrefetchScalarGridSpec(
        num_scalar_prefetch=1,
        grid=(_NT,),
        in_specs=[
            pl.BlockSpec((_TILE, d), lambda i, te: (i, 0)),
            pl.BlockSpec((1, FF, d), lambda i, te: (te[i], 0, 0)),
            pl.BlockSpec((1, FF, d), lambda i, te: (te[i], 0, 0)),
            pl.BlockSpec((1, d, FF), lambda i, te: (te[i], 0, 0)),
        ],
        out_specs=pl.BlockSpec((_TILE, d), lambda i, te: (i, 0)),
    )
    y = pl.pallas_call(
        _expert_body,
        grid_spec=grid_spec,
        out_shape=jax.ShapeDtypeStruct((PAD, d), jnp.float32),
    )(te_arr, gx, W_gate, W_up, W_down)

    out = _make_combine(T, d)(y, p1, p2, wa, wb)
    return out.reshape(b, s, d)


# speedup vs baseline: 1.0780x; 1.0780x over previous
import functools

import jax
import jax.numpy as jnp
from jax import lax
from jax.experimental import pallas as pl
from jax.experimental.pallas import tpu as pltpu
from jax.experimental.pallas import tpu_sc as plsc

_E = 64
_TILE = 32
_NT = 96
_NW = 32
_LANES = 16


def _route_body(x_ref, wr_ref, pos1_ref, pos2_ref, w1_ref, w2_ref, te_ref):
    T = x_ref.shape[0]
    x = x_ref[...]
    logits = lax.dot_general(x, wr_ref[...], (((1,), (1,)), ((), ())),
                             preferred_element_type=jnp.float32)
    m = jnp.max(logits, axis=1, keepdims=True)
    p = jnp.exp(logits - m)
    probs = p / jnp.sum(p, axis=1, keepdims=True)
    lane = lax.broadcasted_iota(jnp.int32, (T, _E), 1)
    m1 = jnp.max(probs, axis=1, keepdims=True)
    i1 = jnp.min(jnp.where(probs == m1, lane, _E), axis=1, keepdims=True)
    probs2 = jnp.where(lane == i1, -jnp.inf, probs)
    m2 = jnp.max(probs2, axis=1, keepdims=True)
    i2 = jnp.min(jnp.where(probs2 == m2, lane, _E), axis=1, keepdims=True)
    s = m1 + m2
    w1_ref[...] = m1 / s
    w2_ref[...] = m2 / s

    M1 = (lane == i1).astype(jnp.float32)
    M2 = (lane == i2).astype(jnp.float32)
    cnt1 = jnp.sum(M1, axis=0, keepdims=True)
    cnt2 = jnp.sum(M2, axis=0, keepdims=True)
    cnt = cnt1 + cnt2
    pcnt = jnp.ceil(cnt / _TILE) * _TILE
    ea = lax.broadcasted_iota(jnp.int32, (_E, _E), 0)
    eb = lax.broadcasted_iota(jnp.int32, (_E, _E), 1)
    U = (ea <= eb).astype(jnp.float32)
    pcum = lax.dot_general(pcnt, U, (((1,), (0,)), ((), ())),
                           preferred_element_type=jnp.float32)
    pex = pcum - pcnt
    ta = lax.broadcasted_iota(jnp.int32, (T, T), 0)
    tb = lax.broadcasted_iota(jnp.int32, (T, T), 1)
    S = (tb < ta).astype(jnp.float32)
    C1 = lax.dot_general(S, M1, (((1,), (0,)), ((), ())),
                         preferred_element_type=jnp.float32)
    C2 = lax.dot_general(S, M2, (((1,), (0,)), ((), ())),
                         preferred_element_type=jnp.float32)
    r1 = jnp.sum(C1 * M1, axis=1, keepdims=True)
    r2 = jnp.sum(C2 * M2, axis=1, keepdims=True)
    pos1 = jnp.sum(pex * M1, axis=1, keepdims=True) + r1
    pos2 = jnp.sum((pex + cnt1) * M2, axis=1, keepdims=True) + r2
    pos1_ref[...] = jnp.round(pos1).astype(jnp.int32)
    pos2_ref[...] = jnp.round(pos2).astype(jnp.int32)

    ts = jnp.round(pex / _TILE).astype(jnp.int32)
    nt = jnp.round(pcnt / _TILE).astype(jnp.int32)
    te_ref[0:8, :] = jnp.broadcast_to(ts, (8, _E))
    te_ref[8:16, :] = jnp.broadcast_to(nt, (8, _E))


def _expert_body(ts_ref, nt_ref, gx_ref, wg_ref, wu_ref, wd_ref, y_ref):
    e = pl.program_id(0)
    wg = wg_ref[0]
    wu = wu_ref[0]
    wd = wd_ref[0]
    start = ts_ref[e]

    def tile_body(k, _):
        row0 = (start + k) * _TILE
        gx = gx_ref[pl.ds(row0, _TILE), :]
        g = lax.dot_general(gx, wg, (((1,), (1,)), ((), ())),
                            preferred_element_type=jnp.float32)
        u = lax.dot_general(gx, wu, (((1,), (1,)), ((), ())),
                            preferred_element_type=jnp.float32)
        h = (g * jax.nn.sigmoid(g)) * u
        y_ref[pl.ds(row0, _TILE), :] = lax.dot_general(
            h, wd, (((1,), (1,)), ((), ())),
            preferred_element_type=jnp.float32)
        return 0

    lax.fori_loop(0, nt_ref[e], tile_body, 0)


def _make_scatter(T, D, PAD):
    ntok = T // _NW
    mesh = plsc.VectorSubcoreMesh(core_axis_name="c", subcore_axis_name="s")

    @functools.partial(
        pl.kernel, mesh=mesh,
        out_type=jax.ShapeDtypeStruct((PAD, D), jnp.float32),
        scratch_types=[
            pltpu.VMEM((ntok,), jnp.int32),
            pltpu.VMEM((ntok,), jnp.int32),
            pltpu.VMEM((ntok, D), jnp.float32),
            pltpu.SemaphoreType.DMA,
            pltpu.SemaphoreType.DMA,
        ],
    )
    def scatter_k(x_hbm, p1_hbm, p2_hbm, gx_hbm, i1v, i2v, rows, sem1, sem2):
        wid = lax.axis_index("s") * 2 + lax.axis_index("c")
        base = wid * ntok
        pltpu.sync_copy(p1_hbm.at[pl.ds(base, ntok)], i1v)
        pltpu.sync_copy(p2_hbm.at[pl.ds(base, ntok)], i2v)
        pltpu.sync_copy(x_hbm.at[pl.ds(base, ntok)], rows)
        c1 = pltpu.async_copy(rows, gx_hbm.at[i1v], sem1)
        c2 = pltpu.async_copy(rows, gx_hbm.at[i2v], sem2)
        c1.wait()
        c2.wait()

    return scatter_k


def _make_combine(T, D):
    ntok = T // _NW
    nchunk = D // _LANES
    mesh = plsc.VectorSubcoreMesh(core_axis_name="c", subcore_axis_name="s")

    @functools.partial(
        pl.kernel, mesh=mesh,
        out_type=jax.ShapeDtypeStruct((T, D), jnp.float32),
        scratch_types=[
            pltpu.VMEM((ntok,), jnp.int32),
            pltpu.VMEM((ntok,), jnp.int32),
            pltpu.VMEM((ntok,), jnp.float32),
            pltpu.VMEM((ntok,), jnp.float32),
            pltpu.VMEM((ntok, D), jnp.float32),
            pltpu.VMEM((ntok, D), jnp.float32),
            pltpu.SemaphoreType.DMA,
            pltpu.SemaphoreType.DMA,
        ],
    )
    def combine_k(y_hbm, p1_hbm, p2_hbm, wa_hbm, wb_hbm, out_hbm,
                  i1v, i2v, w1v, w2v, buf1, buf2, sem1, sem2):
        wid = lax.axis_index("s") * 2 + lax.axis_index("c")
        base = wid * ntok
        pltpu.sync_copy(p1_hbm.at[pl.ds(base, ntok)], i1v)
        pltpu.sync_copy(p2_hbm.at[pl.ds(base, ntok)], i2v)
        pltpu.sync_copy(wa_hbm.at[pl.ds(base, ntok)], w1v)
        pltpu.sync_copy(wb_hbm.at[pl.ds(base, ntok)], w2v)
        c1 = pltpu.async_copy(y_hbm.at[i1v], buf1, sem1)
        c2 = pltpu.async_copy(y_hbm.at[i2v], buf2, sem2)
        c1.wait()
        c2.wait()
        wvec1 = w1v[...]
        wvec2 = w2v[...]
        for row in range(ntok):
            ws1 = jnp.full((_LANES,), wvec1[row], jnp.float32)
            ws2 = jnp.full((_LANES,), wvec2[row], jnp.float32)

            def body(c, _, row=row, ws1=ws1, ws2=ws2):
                a = buf1[row, pl.ds(c * _LANES, _LANES)]
                bb = buf2[row, pl.ds(c * _LANES, _LANES)]
                buf1[row, pl.ds(c * _LANES, _LANES)] = ws1 * a + ws2 * bb
                return 0

            lax.fori_loop(0, nchunk, body, 0)
        pltpu.sync_copy(buf1, out_hbm.at[pl.ds(base, ntok)])

    return combine_k


def kernel(hidden_states, W_router, W_gate, W_up, W_down):
    b, s, d = hidden_states.shape
    x = hidden_states.reshape(-1, d)
    T = x.shape[0]
    E, FF = W_gate.shape[0], W_gate.shape[1]
    PAD = _NT * _TILE

    pos1, pos2, w1, w2, te = pl.pallas_call(
        _route_body,
        grid=(1,),
        in_specs=[
            pl.BlockSpec((T, d), lambda i: (0, 0)),
            pl.BlockSpec((E, d), lambda i: (0, 0)),
        ],
        out_specs=[
            pl.BlockSpec((T, 1), lambda i: (0, 0)),
            pl.BlockSpec((T, 1), lambda i: (0, 0)),
            pl.BlockSpec((T, 1), lambda i: (0, 0)),
            pl.BlockSpec((T, 1), lambda i: (0, 0)),
            pl.BlockSpec((16, E), lambda i: (0, 0)),
        ],
        out_shape=[
            jax.ShapeDtypeStruct((T, 1), jnp.int32),
            jax.ShapeDtypeStruct((T, 1), jnp.int32),
            jax.ShapeDtypeStruct((T, 1), jnp.float32),
            jax.ShapeDtypeStruct((T, 1), jnp.float32),
            jax.ShapeDtypeStruct((16, E), jnp.int32),
        ],
    )(x, W_router)
    p1 = pos1.reshape(T)
    p2 = pos2.reshape(T)
    wa = w1.reshape(T)
    wb = w2.reshape(T)
    ts_arr = te[0]
    nt_arr = te[8]

    gx = _make_scatter(T, d, PAD)(x, p1, p2)

    grid_spec = pltpu.PrefetchScalarGridSpec(
        num_scalar_prefetch=2,
        grid=(E,),
        in_specs=[
            pl.BlockSpec((PAD, d), lambda e, ts, nt: (0, 0)),
            pl.BlockSpec((1, FF, d), lambda e, ts, nt: (e, 0, 0)),
            pl.BlockSpec((1, FF, d), lambda e, ts, nt: (e, 0, 0)),
            pl.BlockSpec((1, d, FF), lambda e, ts, nt: (e, 0, 0)),
        ],
        out_specs=pl.BlockSpec((PAD, d), lambda e, ts, nt: (0, 0)),
    )
    y = pl.pallas_call(
        _expert_body,
        grid_spec=grid_spec,
        out_shape=jax.ShapeDtypeStruct((PAD, d), jnp.float32),
    )(ts_arr, nt_arr, gx, W_gate, W_up, W_down)

    out = _make_combine(T, d)(y, p1, p2, wa, wb)
    return out.reshape(b, s, d)


# speedup vs baseline: 1.0890x; 1.0103x over previous
import functools

import jax
import jax.numpy as jnp
from jax import lax
from jax.experimental import pallas as pl
from jax.experimental.pallas import tpu as pltpu
from jax.experimental.pallas import tpu_sc as plsc

_E = 64
_TILE = 32
_NT = 96
_NW = 32
_LANES = 16


def _route_body(x_ref, wr_ref, pos1_ref, pos2_ref, w1_ref, w2_ref, te_ref):
    T = x_ref.shape[0]
    x = x_ref[...]
    logits = lax.dot_general(x, wr_ref[...], (((1,), (1,)), ((), ())),
                             preferred_element_type=jnp.float32)
    m = jnp.max(logits, axis=1, keepdims=True)
    p = jnp.exp(logits - m)
    probs = p / jnp.sum(p, axis=1, keepdims=True)
    lane = lax.broadcasted_iota(jnp.int32, (T, _E), 1)
    m1 = jnp.max(probs, axis=1, keepdims=True)
    i1 = jnp.min(jnp.where(probs == m1, lane, _E), axis=1, keepdims=True)
    probs2 = jnp.where(lane == i1, -jnp.inf, probs)
    m2 = jnp.max(probs2, axis=1, keepdims=True)
    i2 = jnp.min(jnp.where(probs2 == m2, lane, _E), axis=1, keepdims=True)
    s = m1 + m2
    w1_ref[...] = m1 / s
    w2_ref[...] = m2 / s

    M1 = (lane == i1).astype(jnp.float32)
    M2 = (lane == i2).astype(jnp.float32)
    cnt1 = jnp.sum(M1, axis=0, keepdims=True)
    cnt2 = jnp.sum(M2, axis=0, keepdims=True)
    cnt = cnt1 + cnt2
    pcnt = jnp.ceil(cnt / _TILE) * _TILE
    ea = lax.broadcasted_iota(jnp.int32, (_E, _E), 0)
    eb = lax.broadcasted_iota(jnp.int32, (_E, _E), 1)
    U = (ea <= eb).astype(jnp.float32)
    pcum = lax.dot_general(pcnt, U, (((1,), (0,)), ((), ())),
                           preferred_element_type=jnp.float32)
    pex = pcum - pcnt
    ta = lax.broadcasted_iota(jnp.int32, (T, T), 0)
    tb = lax.broadcasted_iota(jnp.int32, (T, T), 1)
    S = (tb < ta).astype(jnp.float32)
    C1 = lax.dot_general(S, M1, (((1,), (0,)), ((), ())),
                         preferred_element_type=jnp.float32)
    C2 = lax.dot_general(S, M2, (((1,), (0,)), ((), ())),
                         preferred_element_type=jnp.float32)
    r1 = jnp.sum(C1 * M1, axis=1, keepdims=True)
    r2 = jnp.sum(C2 * M2, axis=1, keepdims=True)
    pos1 = jnp.sum(pex * M1, axis=1, keepdims=True) + r1
    pos2 = jnp.sum((pex + cnt1) * M2, axis=1, keepdims=True) + r2
    pos1_ref[...] = jnp.round(pos1).astype(jnp.int32)
    pos2_ref[...] = jnp.round(pos2).astype(jnp.int32)

    ts = jnp.round(pex / _TILE).astype(jnp.int32)
    nt = jnp.round(pcnt / _TILE).astype(jnp.int32)
    te_ref[0:8, :] = jnp.broadcast_to(ts, (8, _E))
    te_ref[8:16, :] = jnp.broadcast_to(nt, (8, _E))


def _expert_body(tab_ref, gx_ref, wg_ref, wu_ref, wd_ref, y_ref):
    e = pl.program_id(0)
    wg = wg_ref[0]
    wu = wu_ref[0]
    wd = wd_ref[0]
    start = tab_ref[0, e]

    def tile_body(k, _):
        row0 = (start + k) * _TILE
        gx = gx_ref[pl.ds(row0, _TILE), :]
        g = lax.dot_general(gx, wg, (((1,), (1,)), ((), ())),
                            preferred_element_type=jnp.float32)
        u = lax.dot_general(gx, wu, (((1,), (1,)), ((), ())),
                            preferred_element_type=jnp.float32)
        h = (g * jax.nn.sigmoid(g)) * u
        y_ref[pl.ds(row0, _TILE), :] = lax.dot_general(
            h, wd, (((1,), (1,)), ((), ())),
            preferred_element_type=jnp.float32)
        return 0

    lax.fori_loop(0, tab_ref[8, e], tile_body, 0)


def _make_scatter(T, D, PAD):
    ntok = T // _NW
    mesh = plsc.VectorSubcoreMesh(core_axis_name="c", subcore_axis_name="s")

    @functools.partial(
        pl.kernel, mesh=mesh,
        out_type=jax.ShapeDtypeStruct((PAD, D), jnp.float32),
        scratch_types=[
            pltpu.VMEM((ntok,), jnp.int32),
            pltpu.VMEM((ntok,), jnp.int32),
            pltpu.VMEM((ntok, D), jnp.float32),
            pltpu.SemaphoreType.DMA,
            pltpu.SemaphoreType.DMA,
        ],
    )
    def scatter_k(x_hbm, p1_hbm, p2_hbm, gx_hbm, i1v, i2v, rows, sem1, sem2):
        wid = lax.axis_index("s") * 2 + lax.axis_index("c")
        base = wid * ntok
        pltpu.sync_copy(p1_hbm.at[pl.ds(base, ntok)], i1v)
        pltpu.sync_copy(p2_hbm.at[pl.ds(base, ntok)], i2v)
        pltpu.sync_copy(x_hbm.at[pl.ds(base, ntok)], rows)
        c1 = pltpu.async_copy(rows, gx_hbm.at[i1v], sem1)
        c2 = pltpu.async_copy(rows, gx_hbm.at[i2v], sem2)
        c1.wait()
        c2.wait()

    return scatter_k


def _make_combine(T, D):
    ntok = T // _NW
    nchunk = D // _LANES
    mesh = plsc.VectorSubcoreMesh(core_axis_name="c", subcore_axis_name="s")

    @functools.partial(
        pl.kernel, mesh=mesh,
        out_type=jax.ShapeDtypeStruct((T, D), jnp.float32),
        scratch_types=[
            pltpu.VMEM((ntok,), jnp.int32),
            pltpu.VMEM((ntok,), jnp.int32),
            pltpu.VMEM((ntok,), jnp.float32),
            pltpu.VMEM((ntok,), jnp.float32),
            pltpu.VMEM((ntok, D), jnp.float32),
            pltpu.VMEM((ntok, D), jnp.float32),
            pltpu.SemaphoreType.DMA,
            pltpu.SemaphoreType.DMA,
        ],
    )
    def combine_k(y_hbm, p1_hbm, p2_hbm, wa_hbm, wb_hbm, out_hbm,
                  i1v, i2v, w1v, w2v, buf1, buf2, sem1, sem2):
        wid = lax.axis_index("s") * 2 + lax.axis_index("c")
        base = wid * ntok
        pltpu.sync_copy(p1_hbm.at[pl.ds(base, ntok)], i1v)
        pltpu.sync_copy(p2_hbm.at[pl.ds(base, ntok)], i2v)
        pltpu.sync_copy(wa_hbm.at[pl.ds(base, ntok)], w1v)
        pltpu.sync_copy(wb_hbm.at[pl.ds(base, ntok)], w2v)
        c1 = pltpu.async_copy(y_hbm.at[i1v], buf1, sem1)
        c2 = pltpu.async_copy(y_hbm.at[i2v], buf2, sem2)
        c1.wait()
        c2.wait()
        wvec1 = w1v[...]
        wvec2 = w2v[...]
        for row in range(ntok):
            ws1 = jnp.full((_LANES,), wvec1[row], jnp.float32)
            ws2 = jnp.full((_LANES,), wvec2[row], jnp.float32)

            def body(c, _, row=row, ws1=ws1, ws2=ws2):
                a = buf1[row, pl.ds(c * _LANES, _LANES)]
                bb = buf2[row, pl.ds(c * _LANES, _LANES)]
                buf1[row, pl.ds(c * _LANES, _LANES)] = ws1 * a + ws2 * bb
                return 0

            lax.fori_loop(0, nchunk, body, 0)
        pltpu.sync_copy(buf1, out_hbm.at[pl.ds(base, ntok)])

    return combine_k


def kernel(hidden_states, W_router, W_gate, W_up, W_down):
    b, s, d = hidden_states.shape
    x = hidden_states.reshape(-1, d)
    T = x.shape[0]
    E, FF = W_gate.shape[0], W_gate.shape[1]
    PAD = _NT * _TILE

    pos1, pos2, w1, w2, te = pl.pallas_call(
        _route_body,
        grid=(1,),
        in_specs=[
            pl.BlockSpec((T, d), lambda i: (0, 0)),
            pl.BlockSpec((E, d), lambda i: (0, 0)),
        ],
        out_specs=[
            pl.BlockSpec((T, 1), lambda i: (0, 0)),
            pl.BlockSpec((T, 1), lambda i: (0, 0)),
            pl.BlockSpec((T, 1), lambda i: (0, 0)),
            pl.BlockSpec((T, 1), lambda i: (0, 0)),
            pl.BlockSpec((16, E), lambda i: (0, 0)),
        ],
        out_shape=[
            jax.ShapeDtypeStruct((T, 1), jnp.int32),
            jax.ShapeDtypeStruct((T, 1), jnp.int32),
            jax.ShapeDtypeStruct((T, 1), jnp.float32),
            jax.ShapeDtypeStruct((T, 1), jnp.float32),
            jax.ShapeDtypeStruct((16, E), jnp.int32),
        ],
    )(x, W_router)
    p1 = pos1.reshape(T)
    p2 = pos2.reshape(T)
    wa = w1.reshape(T)
    wb = w2.reshape(T)
    gx = _make_scatter(T, d, PAD)(x, p1, p2)

    y = pl.pallas_call(
        _expert_body,
        grid=(E,),
        in_specs=[
            pl.BlockSpec(memory_space=pltpu.SMEM),
            pl.BlockSpec((PAD, d), lambda e: (0, 0)),
            pl.BlockSpec((1, FF, d), lambda e: (e, 0, 0)),
            pl.BlockSpec((1, FF, d), lambda e: (e, 0, 0)),
            pl.BlockSpec((1, d, FF), lambda e: (e, 0, 0)),
        ],
        out_specs=pl.BlockSpec((PAD, d), lambda e: (0, 0)),
        out_shape=jax.ShapeDtypeStruct((PAD, d), jnp.float32),
    )(te, gx, W_gate, W_up, W_down)

    out = _make_combine(T, d)(y, p1, p2, wa, wb)
    return out.reshape(b, s, d)


# speedup vs baseline: 1.2472x; 1.1453x over previous
import functools

import jax
import jax.numpy as jnp
from jax import lax
from jax.experimental import pallas as pl
from jax.experimental.pallas import tpu as pltpu
from jax.experimental.pallas import tpu_sc as plsc

_E = 64
_TILE = 32
_NT = 96
_NW = 32
_LANES = 16


def _route_body(x_ref, wr_ref, pos1_ref, pos2_ref, w1_ref, w2_ref, te_ref):
    T = x_ref.shape[0]
    x = x_ref[...]
    logits = lax.dot_general(x, wr_ref[...], (((1,), (1,)), ((), ())),
                             preferred_element_type=jnp.float32)
    m = jnp.max(logits, axis=1, keepdims=True)
    p = jnp.exp(logits - m)
    probs = p / jnp.sum(p, axis=1, keepdims=True)
    lane = lax.broadcasted_iota(jnp.int32, (T, _E), 1)
    m1 = jnp.max(probs, axis=1, keepdims=True)
    i1 = jnp.min(jnp.where(probs == m1, lane, _E), axis=1, keepdims=True)
    probs2 = jnp.where(lane == i1, -jnp.inf, probs)
    m2 = jnp.max(probs2, axis=1, keepdims=True)
    i2 = jnp.min(jnp.where(probs2 == m2, lane, _E), axis=1, keepdims=True)
    s = m1 + m2
    w1_ref[...] = m1 / s
    w2_ref[...] = m2 / s

    M1 = (lane == i1).astype(jnp.float32)
    M2 = (lane == i2).astype(jnp.float32)
    cnt1 = jnp.sum(M1, axis=0, keepdims=True)
    cnt2 = jnp.sum(M2, axis=0, keepdims=True)
    cnt = cnt1 + cnt2
    pcnt = jnp.ceil(cnt / _TILE) * _TILE
    ea = lax.broadcasted_iota(jnp.int32, (_E, _E), 0)
    eb = lax.broadcasted_iota(jnp.int32, (_E, _E), 1)
    U = (ea <= eb).astype(jnp.float32)
    pcum = lax.dot_general(pcnt, U, (((1,), (0,)), ((), ())),
                           preferred_element_type=jnp.float32)
    pex = pcum - pcnt
    ta = lax.broadcasted_iota(jnp.int32, (T, T), 0)
    tb = lax.broadcasted_iota(jnp.int32, (T, T), 1)
    S = (tb < ta).astype(jnp.float32)
    C1 = lax.dot_general(S, M1, (((1,), (0,)), ((), ())),
                         preferred_element_type=jnp.float32)
    C2 = lax.dot_general(S, M2, (((1,), (0,)), ((), ())),
                         preferred_element_type=jnp.float32)
    r1 = jnp.sum(C1 * M1, axis=1, keepdims=True)
    r2 = jnp.sum(C2 * M2, axis=1, keepdims=True)
    pos1 = jnp.sum(pex * M1, axis=1, keepdims=True) + r1
    pos2 = jnp.sum((pex + cnt1) * M2, axis=1, keepdims=True) + r2
    pos1_ref[...] = jnp.round(pos1).astype(jnp.int32)
    pos2_ref[...] = jnp.round(pos2).astype(jnp.int32)

    ts = jnp.round(pex / _TILE).astype(jnp.int32)
    nt = jnp.round(pcnt / _TILE).astype(jnp.int32)
    te_ref[0:8, :] = jnp.broadcast_to(ts, (8, _E))
    te_ref[8:16, :] = jnp.broadcast_to(nt, (8, _E))


_RING = 3


def _expert_body(tab_ref, gx_ref, wg_hbm, wu_hbm, wd_hbm, y_ref,
                 wgb, wub, wdb, sems):
    K = _RING

    def issue(eidx, b):
        pltpu.make_async_copy(wg_hbm.at[eidx], wgb.at[b], sems.at[b, 0]).start()
        pltpu.make_async_copy(wu_hbm.at[eidx], wub.at[b], sems.at[b, 1]).start()
        pltpu.make_async_copy(wd_hbm.at[eidx], wdb.at[b], sems.at[b, 2]).start()

    for b in range(K):
        issue(b, b)

    def step(e, _):
        b = lax.rem(e, K)
        pltpu.make_async_copy(wg_hbm.at[e], wgb.at[b], sems.at[b, 0]).wait()
        pltpu.make_async_copy(wu_hbm.at[e], wub.at[b], sems.at[b, 1]).wait()
        pltpu.make_async_copy(wd_hbm.at[e], wdb.at[b], sems.at[b, 2]).wait()
        wg = wgb[b]
        wu = wub[b]
        wd = wdb[b]
        start = tab_ref[0, e]

        def tile_body(k, _):
            row0 = (start + k) * _TILE
            gx = gx_ref[pl.ds(row0, _TILE), :]
            g = lax.dot_general(gx, wg, (((1,), (1,)), ((), ())),
                                preferred_element_type=jnp.float32)
            u = lax.dot_general(gx, wu, (((1,), (1,)), ((), ())),
                                preferred_element_type=jnp.float32)
            h = (g * jax.nn.sigmoid(g)) * u
            y_ref[pl.ds(row0, _TILE), :] = lax.dot_general(
                h, wd, (((1,), (1,)), ((), ())),
                preferred_element_type=jnp.float32)
            return 0

        lax.fori_loop(0, tab_ref[8, e], tile_body, 0)

        @pl.when(e + K < _E)
        def _():
            issue(e + K, b)

        return 0

    lax.fori_loop(0, _E, step, 0)


def _make_scatter(T, D, PAD):
    ntok = T // _NW
    mesh = plsc.VectorSubcoreMesh(core_axis_name="c", subcore_axis_name="s")

    @functools.partial(
        pl.kernel, mesh=mesh,
        out_type=jax.ShapeDtypeStruct((PAD, D), jnp.float32),
        scratch_types=[
            pltpu.VMEM((ntok,), jnp.int32),
            pltpu.VMEM((ntok,), jnp.int32),
            pltpu.VMEM((ntok, D), jnp.float32),
            pltpu.SemaphoreType.DMA,
            pltpu.SemaphoreType.DMA,
        ],
    )
    def scatter_k(x_hbm, p1_hbm, p2_hbm, gx_hbm, i1v, i2v, rows, sem1, sem2):
        wid = lax.axis_index("s") * 2 + lax.axis_index("c")
        base = wid * ntok
        pltpu.sync_copy(p1_hbm.at[pl.ds(base, ntok)], i1v)
        pltpu.sync_copy(p2_hbm.at[pl.ds(base, ntok)], i2v)
        pltpu.sync_copy(x_hbm.at[pl.ds(base, ntok)], rows)
        c1 = pltpu.async_copy(rows, gx_hbm.at[i1v], sem1)
        c2 = pltpu.async_copy(rows, gx_hbm.at[i2v], sem2)
        c1.wait()
        c2.wait()

    return scatter_k


def _make_combine(T, D):
    ntok = T // _NW
    nchunk = D // _LANES
    mesh = plsc.VectorSubcoreMesh(core_axis_name="c", subcore_axis_name="s")

    @functools.partial(
        pl.kernel, mesh=mesh,
        out_type=jax.ShapeDtypeStruct((T, D), jnp.float32),
        scratch_types=[
            pltpu.VMEM((ntok,), jnp.int32),
            pltpu.VMEM((ntok,), jnp.int32),
            pltpu.VMEM((ntok,), jnp.float32),
            pltpu.VMEM((ntok,), jnp.float32),
            pltpu.VMEM((ntok, D), jnp.float32),
            pltpu.VMEM((ntok, D), jnp.float32),
            pltpu.SemaphoreType.DMA,
            pltpu.SemaphoreType.DMA,
        ],
    )
    def combine_k(y_hbm, p1_hbm, p2_hbm, wa_hbm, wb_hbm, out_hbm,
                  i1v, i2v, w1v, w2v, buf1, buf2, sem1, sem2):
        wid = lax.axis_index("s") * 2 + lax.axis_index("c")
        base = wid * ntok
        pltpu.sync_copy(p1_hbm.at[pl.ds(base, ntok)], i1v)
        pltpu.sync_copy(p2_hbm.at[pl.ds(base, ntok)], i2v)
        pltpu.sync_copy(wa_hbm.at[pl.ds(base, ntok)], w1v)
        pltpu.sync_copy(wb_hbm.at[pl.ds(base, ntok)], w2v)
        c1 = pltpu.async_copy(y_hbm.at[i1v], buf1, sem1)
        c2 = pltpu.async_copy(y_hbm.at[i2v], buf2, sem2)
        c1.wait()
        c2.wait()
        wvec1 = w1v[...]
        wvec2 = w2v[...]
        for row in range(ntok):
            ws1 = jnp.full((_LANES,), wvec1[row], jnp.float32)
            ws2 = jnp.full((_LANES,), wvec2[row], jnp.float32)

            def body(c, _, row=row, ws1=ws1, ws2=ws2):
                a = buf1[row, pl.ds(c * _LANES, _LANES)]
                bb = buf2[row, pl.ds(c * _LANES, _LANES)]
                buf1[row, pl.ds(c * _LANES, _LANES)] = ws1 * a + ws2 * bb
                return 0

            lax.fori_loop(0, nchunk, body, 0)
        pltpu.sync_copy(buf1, out_hbm.at[pl.ds(base, ntok)])

    return combine_k


def kernel(hidden_states, W_router, W_gate, W_up, W_down):
    b, s, d = hidden_states.shape
    x = hidden_states.reshape(-1, d)
    T = x.shape[0]
    E, FF = W_gate.shape[0], W_gate.shape[1]
    PAD = _NT * _TILE

    pos1, pos2, w1, w2, te = pl.pallas_call(
        _route_body,
        grid=(1,),
        in_specs=[
            pl.BlockSpec((T, d), lambda i: (0, 0)),
            pl.BlockSpec((E, d), lambda i: (0, 0)),
        ],
        out_specs=[
            pl.BlockSpec((T, 1), lambda i: (0, 0)),
            pl.BlockSpec((T, 1), lambda i: (0, 0)),
            pl.BlockSpec((T, 1), lambda i: (0, 0)),
            pl.BlockSpec((T, 1), lambda i: (0, 0)),
            pl.BlockSpec((16, E), lambda i: (0, 0)),
        ],
        out_shape=[
            jax.ShapeDtypeStruct((T, 1), jnp.int32),
            jax.ShapeDtypeStruct((T, 1), jnp.int32),
            jax.ShapeDtypeStruct((T, 1), jnp.float32),
            jax.ShapeDtypeStruct((T, 1), jnp.float32),
            jax.ShapeDtypeStruct((16, E), jnp.int32),
        ],
    )(x, W_router)
    p1 = pos1.reshape(T)
    p2 = pos2.reshape(T)
    wa = w1.reshape(T)
    wb = w2.reshape(T)
    gx = _make_scatter(T, d, PAD)(x, p1, p2)

    y = pl.pallas_call(
        _expert_body,
        grid=(1,),
        in_specs=[
            pl.BlockSpec(memory_space=pltpu.SMEM),
            pl.BlockSpec((PAD, d), lambda i: (0, 0)),
            pl.BlockSpec(memory_space=pl.ANY),
            pl.BlockSpec(memory_space=pl.ANY),
            pl.BlockSpec(memory_space=pl.ANY),
        ],
        out_specs=pl.BlockSpec((PAD, d), lambda i: (0, 0)),
        out_shape=jax.ShapeDtypeStruct((PAD, d), jnp.float32),
        scratch_shapes=[
            pltpu.VMEM((_RING, FF, d), jnp.float32),
            pltpu.VMEM((_RING, FF, d), jnp.float32),
            pltpu.VMEM((_RING, d, FF), jnp.float32),
            pltpu.SemaphoreType.DMA((_RING, 3)),
        ],
    )(te, gx, W_gate, W_up, W_down)

    out = _make_combine(T, d)(y, p1, p2, wa, wb)
    return out.reshape(b, s, d)
